# baseline (speedup 1.0000x reference)
import functools

import jax
import jax.numpy as jnp
from jax import lax
from jax.experimental import pallas as pl
from jax.experimental.pallas import tpu as pltpu
from jax.experimental.pallas import tpu_sc as plsc

NW = 32
C = 128
K = 2


def _tail_sizes(rem):
    out = []
    while rem:
        t = min(rem, C)
        out.append(t)
        rem -= t
    return out


def _make_gather(N, E, D, base, Ec):
    epw = Ec // NW
    nfull = (epw // C) & ~1
    npairs = nfull // 2
    tails = _tail_sizes(epw - nfull * C)
    assert npairs >= 1
    mesh = plsc.VectorSubcoreMesh(core_axis_name="c", subcore_axis_name="s")

    @functools.partial(
        pl.kernel,
        out_type=jax.ShapeDtypeStruct((Ec, D), jnp.float32),
        mesh=mesh,
        scratch_types=[
            pltpu.VMEM((C,), jnp.int32),
            pltpu.VMEM((C,), jnp.int32),
            pltpu.VMEM((C,), jnp.int32),
            pltpu.VMEM((C,), jnp.int32),
            pltpu.VMEM((C, D), jnp.float32),
            pltpu.VMEM((C, D), jnp.float32),
            pltpu.VMEM((C, D), jnp.float32),
            pltpu.VMEM((C, D), jnp.float32),
            pltpu.SemaphoreType.DMA,
            pltpu.SemaphoreType.DMA,
            pltpu.SemaphoreType.DMA,
            pltpu.SemaphoreType.DMA,
            pltpu.SemaphoreType.DMA,
            pltpu.SemaphoreType.DMA,
        ],
    )
    def gather_k(p_hbm, q_hbm, src_hbm, dst_hbm, g_hbm,
                 sidx0, sidx1, didx0, didx1, bs0, bs1, bd0, bd1,
                 semA0, semA1, semB0, semB1, semC0, semC1):
        SIDX = [sidx0, sidx1]
        DIDX = [didx0, didx1]
        BS = [bs0, bs1]
        BD = [bd0, bd1]
        SEMA = [semA0, semA1]
        SEMB = [semB0, semB1]
        SEMC = [semC0, semC1]
        wid = lax.axis_index("s") * 2 + lax.axis_index("c")
        lbase = wid * epw
        gbase = base + lbase

        def a_descs(b, j):
            off = gbase + j * C
            return (pltpu.make_async_copy(src_hbm.at[pl.ds(off, C)], SIDX[b], SEMA[b]),
                    pltpu.make_async_copy(dst_hbm.at[pl.ds(off, C)], DIDX[b], SEMA[b]))

        def b_descs(b):
            return (pltpu.make_async_copy(p_hbm.at[SIDX[b]], BS[b], SEMB[b]),
                    pltpu.make_async_copy(q_hbm.at[DIDX[b]], BD[b], SEMB[b]))

        def c_descs(b, j):
            off = lbase + j * C
            return (pltpu.make_async_copy(BS[b], g_hbm.at[pl.ds(off, C)], SEMC[b]),)

        def add_rows(dst_ref, src_ref, nrows):
            def row(i, carry):
                for k in range(D // 16):
                    plsc.addupdate(dst_ref.at[i, pl.ds(16 * k, 16)],
                                   src_ref[i, pl.ds(16 * k, 16)])
                return carry
            lax.fori_loop(0, nrows, row, 0)

        def start(descs):
            for d in descs:
                d.start()

        def wait(descs):
            for d in descs:
                d.wait()

        start(a_descs(0, 0))
        start(a_descs(1, 1))

        def body(j2, carry):
            j0 = 2 * j2
            for b in (0, 1):
                wait(a_descs(b, j0 + b))

                @pl.when(j2 > 0)
                def _(b=b, j0=j0):
                    wait(c_descs(b, j0 + b - 2))

                start(b_descs(b))
            for b in (0, 1):
                wait(b_descs(b))
                add_rows(BS[b], BD[b], C)
                start(c_descs(b, j0 + b))

                @pl.when(j2 < npairs - 1)
                def _(b=b, j0=j0):
                    start(a_descs(b, j0 + 2 + b))

            return carry

        lax.fori_loop(0, npairs, body, 0)
        wait(c_descs(0, nfull - 2))
        wait(c_descs(1, nfull - 1))

        toff = nfull * C
        for t in tails:
            pltpu.sync_copy(src_hbm.at[pl.ds(gbase + toff, t)],
                            sidx0.at[pl.ds(0, t)])
            pltpu.sync_copy(dst_hbm.at[pl.ds(gbase + toff, t)],
                            didx0.at[pl.ds(0, t)])
            tb = (pltpu.make_async_copy(
                      p_hbm.at[sidx0.at[pl.ds(0, t)]], bs0.at[pl.ds(0, t)], semB0),
                  pltpu.make_async_copy(
                      q_hbm.at[didx0.at[pl.ds(0, t)]], bd0.at[pl.ds(0, t)], semB1))
            start(tb)
            wait(tb)
            add_rows(bs0, bd0, t)
            pltpu.sync_copy(bs0.at[pl.ds(0, t)],
                            g_hbm.at[pl.ds(lbase + toff, t)])
            toff += t

    return gather_k


def _make_scatter(NPAD, E, D, base, Ec):
    epw = Ec // NW
    nfull = (epw // C) & ~1
    npairs = nfull // 2
    tails = _tail_sizes(epw - nfull * C)
    assert npairs >= 1
    rpt = NPAD // 16
    mesh = plsc.VectorSubcoreMesh(core_axis_name="c", subcore_axis_name="s")

    tail_scratch = [pltpu.VMEM((t,), jnp.int32) for t in tails]

    @functools.partial(
        pl.kernel,
        out_type=jax.ShapeDtypeStruct((2, NPAD, D), jnp.float32),
        mesh=mesh,
        scratch_types=[
            pltpu.VMEM((C,), jnp.int32),
            pltpu.VMEM((C,), jnp.int32),
            pltpu.VMEM((C, D), jnp.float32),
            pltpu.VMEM((C, D), jnp.float32),
            pltpu.VMEM_SHARED((NPAD, D), jnp.float32),
            pltpu.SemaphoreType.DMA,
            pltpu.SemaphoreType.DMA,
        ] + tail_scratch,
    )
    def scatter_k(em_hbm, dst_hbm, zeros_hbm, out_hbm,
                  idx0, idx1, rows0, rows1, agg, semA0, semA1, *tscratch):
        IDX = [idx0, idx1]
        ROWS = [rows0, rows1]
        SEMA = [semA0, semA1]
        c = lax.axis_index("c")
        s = lax.axis_index("s")
        wid = s * 2 + c
        lbase = wid * epw
        gbase = base + lbase
        pltpu.sync_copy(zeros_hbm.at[pl.ds(s * rpt, rpt)],
                        agg.at[pl.ds(s * rpt, rpt)])
        plsc.subcore_barrier()

        def a_descs(b, j):
            return (pltpu.make_async_copy(
                        dst_hbm.at[pl.ds(gbase + j * C, C)], IDX[b], SEMA[b]),
                    pltpu.make_async_copy(
                        em_hbm.at[pl.ds(lbase + j * C, C)], ROWS[b], SEMA[b]))

        for b in (0, 1):
            for d in a_descs(b, b):
                d.start()

        def body(j2, carry):
            j0 = 2 * j2
            for b in (0, 1):
                for d in a_descs(b, j0 + b):
                    d.wait()
                pltpu.sync_copy(ROWS[b], agg.at[IDX[b]], add=True)

                @pl.when(j2 < npairs - 1)
                def _(b=b, j0=j0):
                    for d in a_descs(b, j0 + 2 + b):
                        d.start()

            return carry

        lax.fori_loop(0, npairs, body, 0)

        toff = nfull * C
        for i, t in enumerate(tails):
            tidx = tscratch[i]
            pltpu.sync_copy(dst_hbm.at[pl.ds(gbase + toff, t)], tidx)
            pltpu.sync_copy(em_hbm.at[pl.ds(lbase + toff, t)],
                            rows0.at[pl.ds(0, t)])
            pltpu.sync_copy(rows0.at[pl.ds(0, t)], agg.at[tidx], add=True)
            toff += t

        plsc.subcore_barrier()
        pltpu.sync_copy(agg.at[pl.ds(s * rpt, rpt)],
                        out_hbm.at[c].at[pl.ds(s * rpt, rpt)])

    return scatter_k


def _pre_body(x_ref, wb_ref, wc_ref, b1_ref, p_ref, q_ref):
    xb = x_ref[...]
    p_ref[...] = jnp.dot(xb, wb_ref[...], preferred_element_type=jnp.float32)
    q_ref[...] = (jnp.dot(xb, wc_ref[...], preferred_element_type=jnp.float32)
                  + b1_ref[...])


def _edge_math(ea_ref, g_ref, wa_ref, w2_ref, b2_ref, eo_ref, em_ref):
    ea = ea_ref[...]
    h = jnp.maximum(
        jnp.dot(ea, wa_ref[...], preferred_element_type=jnp.float32)
        + g_ref[...], 0.0)
    em = jnp.dot(h, w2_ref[...], preferred_element_type=jnp.float32) + b2_ref[...]
    em_ref[...] = em
    eo_ref[...] = ea + em


def _edge_body(ea_ref, g_ref, wa_ref, w2_ref, b2_ref, eo_ref, em_ref):
    _edge_math(ea_ref, g_ref, wa_ref, w2_ref, b2_ref, eo_ref, em_ref)


def _edge_body_alias(ea_ref, g_ref, wa_ref, w2_ref, b2_ref,
                     eo_prev_ref, eo_ref, em_ref):
    del eo_prev_ref
    _edge_math(ea_ref, g_ref, wa_ref, w2_ref, b2_ref, eo_ref, em_ref)


def _make_node_body(n_parts):
    def body(*refs):
        x_ref = refs[0]
        parts = refs[1:1 + n_parts]
        w1x_ref, w1a_ref, b1_ref, w2_ref, b2_ref, xo_ref = refs[1 + n_parts:]
        xb = x_ref[...]
        agg = parts[0][0]
        for r in parts[1:]:
            agg = agg + r[0]
        h = jnp.maximum(
            jnp.dot(xb, w1x_ref[...], preferred_element_type=jnp.float32)
            + jnp.dot(agg, w1a_ref[...], preferred_element_type=jnp.float32)
            + b1_ref[...], 0.0)
        xo_ref[...] = (xb
                       + jnp.dot(h, w2_ref[...], preferred_element_type=jnp.float32)
                       + b2_ref[...])
    return body


def _fixed(shape):
    return pl.BlockSpec(shape, lambda i: (0,) * len(shape))


def kernel(x, edge_attr, edge_index, We1, be1, We2, be2, Wn1, bn1, Wn2, bn2):
    N, D = x.shape
    E = edge_attr.shape[0]
    src = edge_index[0]
    dst = edge_index[1]
    b1 = be1.reshape(1, D)
    b2 = be2.reshape(1, D)
    nb1 = bn1.reshape(1, D)
    nb2 = bn2.reshape(1, D)

    BN = 1000
    P, Q = pl.pallas_call(
        _pre_body,
        grid=(N // BN,),
        in_specs=[
            pl.BlockSpec((BN, D), lambda i: (i, 0)),
            _fixed((D, D)),
            _fixed((D, D)),
            _fixed((1, D)),
        ],
        out_specs=[
            pl.BlockSpec((BN, D), lambda i: (i, 0)),
            pl.BlockSpec((BN, D), lambda i: (i, 0)),
        ],
        out_shape=[
            jax.ShapeDtypeStruct((N, D), jnp.float32),
            jax.ShapeDtypeStruct((N, D), jnp.float32),
        ],
    )(x, We1[D:2 * D], We1[2 * D:3 * D], b1)

    NPAD = ((N + 127) // 128) * 128
    zeros = jnp.zeros((NPAD, D), jnp.float32)
    BE = 3200
    sizes = [121600, E - 121600] if K == 2 else [E // K] * K
    bases = [sum(sizes[:k]) for k in range(K)]

    Gs = [_make_gather(N, E, D, bases[k], sizes[k])(P, Q, src, dst)
          for k in range(K)]

    eo = None
    EMs = []
    for k in range(K):
        boff = bases[k] // BE
        nblk = sizes[k] // BE
        ea_spec = pl.BlockSpec((BE, D), lambda i, boff=boff: (i + boff, 0))
        eo_spec = pl.BlockSpec((BE, D), lambda i, boff=boff: (i + boff, 0))
        common_in = [
            ea_spec,
            pl.BlockSpec((BE, D), lambda i: (i, 0)),
            _fixed((D, D)),
            _fixed((D, D)),
            _fixed((1, D)),
        ]
        out_shape = [
            jax.ShapeDtypeStruct((E, D), jnp.float32),
            jax.ShapeDtypeStruct((sizes[k], D), jnp.float32),
        ]
        out_specs = [eo_spec, pl.BlockSpec((BE, D), lambda i: (i, 0))]
        if k == 0:
            eo, em = pl.pallas_call(
                _edge_body,
                grid=(nblk,),
                in_specs=common_in,
                out_specs=out_specs,
                out_shape=out_shape,
            )(edge_attr, Gs[k], We1[:D], We2, b2)
        else:
            eo, em = pl.pallas_call(
                _edge_body_alias,
                grid=(nblk,),
                in_specs=common_in + [pl.BlockSpec(memory_space=pl.ANY)],
                out_specs=out_specs,
                out_shape=out_shape,
                input_output_aliases={5: 0},
            )(edge_attr, Gs[k], We1[:D], We2, b2, eo)
        EMs.append(em)

    partials = [_make_scatter(NPAD, E, D, bases[k], sizes[k])(EMs[k], dst, zeros)
                for k in range(K)]

    n_parts = 2 * K
    part_args = []
    part_specs = []
    for par in partials:
        for c in (0, 1):
            part_args.append(par)
            part_specs.append(pl.BlockSpec((1, BN, D), lambda i, c=c: (c, i, 0)))
    XO = pl.pallas_call(
        _make_node_body(n_parts),
        grid=(N // BN,),
        in_specs=[pl.BlockSpec((BN, D), lambda i: (i, 0))] + part_specs + [
            _fixed((D, D)),
            _fixed((D, D)),
            _fixed((1, D)),
            _fixed((D, D)),
            _fixed((1, D)),
        ],
        out_specs=pl.BlockSpec((BN, D), lambda i: (i, 0)),
        out_shape=jax.ShapeDtypeStruct((N, D), jnp.float32),
    )(x, *part_args, Wn1[:D], Wn1[D:], nb1, Wn2, nb2)

    return (XO, eo)


# speedup vs baseline: 1.0142x; 1.0142x over previous
import functools

import jax
import jax.numpy as jnp
from jax import lax
from jax.experimental import pallas as pl
from jax.experimental.pallas import tpu as pltpu
from jax.experimental.pallas import tpu_sc as plsc

NW = 32
C = 128
K = 2


def _tail_sizes(rem):
    out = []
    while rem:
        t = min(rem, C)
        out.append(t)
        rem -= t
    return out


def _make_gather(N, E, D, base, Ec):
    epw = Ec // NW
    nfull = (epw // C) & ~1
    npairs = nfull // 2
    tails = _tail_sizes(epw - nfull * C)
    assert npairs >= 1
    mesh = plsc.VectorSubcoreMesh(core_axis_name="c", subcore_axis_name="s")

    @functools.partial(
        pl.kernel,
        out_type=jax.ShapeDtypeStruct((Ec, D), jnp.float32),
        mesh=mesh,
        scratch_types=[
            pltpu.VMEM((C,), jnp.int32),
            pltpu.VMEM((C,), jnp.int32),
            pltpu.VMEM((C,), jnp.int32),
            pltpu.VMEM((C,), jnp.int32),
            pltpu.VMEM((C, D), jnp.float32),
            pltpu.VMEM((C, D), jnp.float32),
            pltpu.VMEM((C, D), jnp.float32),
            pltpu.VMEM((C, D), jnp.float32),
            pltpu.SemaphoreType.DMA,
            pltpu.SemaphoreType.DMA,
            pltpu.SemaphoreType.DMA,
            pltpu.SemaphoreType.DMA,
            pltpu.SemaphoreType.DMA,
            pltpu.SemaphoreType.DMA,
        ],
    )
    def gather_k(p_hbm, q_hbm, src_hbm, dst_hbm, g_hbm,
                 sidx0, sidx1, didx0, didx1, bs0, bs1, bd0, bd1,
                 semA0, semA1, semB0, semB1, semC0, semC1):
        SIDX = [sidx0, sidx1]
        DIDX = [didx0, didx1]
        BS = [bs0, bs1]
        BD = [bd0, bd1]
        SEMA = [semA0, semA1]
        SEMB = [semB0, semB1]
        SEMC = [semC0, semC1]
        wid = lax.axis_index("s") * 2 + lax.axis_index("c")
        lbase = wid * epw
        gbase = base + lbase

        def a_descs(b, j):
            off = gbase + j * C
            return (pltpu.make_async_copy(src_hbm.at[pl.ds(off, C)], SIDX[b], SEMA[b]),
                    pltpu.make_async_copy(dst_hbm.at[pl.ds(off, C)], DIDX[b], SEMA[b]))

        def b_descs(b):
            return (pltpu.make_async_copy(p_hbm.at[SIDX[b]], BS[b], SEMB[b]),
                    pltpu.make_async_copy(q_hbm.at[DIDX[b]], BD[b], SEMB[b]))

        def c_descs(b, j):
            off = lbase + j * C
            return (pltpu.make_async_copy(BS[b], g_hbm.at[pl.ds(off, C)], SEMC[b]),)

        def add_rows(dst_ref, src_ref, nrows):
            def row(i, carry):
                for k in range(D // 16):
                    plsc.addupdate(dst_ref.at[i, pl.ds(16 * k, 16)],
                                   src_ref[i, pl.ds(16 * k, 16)])
                return carry
            lax.fori_loop(0, nrows, row, 0)

        def start(descs):
            for d in descs:
                d.start()

        def wait(descs):
            for d in descs:
                d.wait()

        start(a_descs(0, 0))
        start(a_descs(1, 1))

        def body(j2, carry):
            j0 = 2 * j2
            for b in (0, 1):
                wait(a_descs(b, j0 + b))

                @pl.when(j2 > 0)
                def _(b=b, j0=j0):
                    wait(c_descs(b, j0 + b - 2))

                start(b_descs(b))
            for b in (0, 1):
                wait(b_descs(b))
                add_rows(BS[b], BD[b], C)
                start(c_descs(b, j0 + b))

                @pl.when(j2 < npairs - 1)
                def _(b=b, j0=j0):
                    start(a_descs(b, j0 + 2 + b))

            return carry

        lax.fori_loop(0, npairs, body, 0)
        wait(c_descs(0, nfull - 2))
        wait(c_descs(1, nfull - 1))

        toff = nfull * C
        for t in tails:
            pltpu.sync_copy(src_hbm.at[pl.ds(gbase + toff, t)],
                            sidx0.at[pl.ds(0, t)])
            pltpu.sync_copy(dst_hbm.at[pl.ds(gbase + toff, t)],
                            didx0.at[pl.ds(0, t)])
            tb = (pltpu.make_async_copy(
                      p_hbm.at[sidx0.at[pl.ds(0, t)]], bs0.at[pl.ds(0, t)], semB0),
                  pltpu.make_async_copy(
                      q_hbm.at[didx0.at[pl.ds(0, t)]], bd0.at[pl.ds(0, t)], semB1))
            start(tb)
            wait(tb)
            add_rows(bs0, bd0, t)
            pltpu.sync_copy(bs0.at[pl.ds(0, t)],
                            g_hbm.at[pl.ds(lbase + toff, t)])
            toff += t

    return gather_k


def _make_scatter(NPAD, E, D, base, Ec):
    epw = Ec // NW
    nfull = (epw // C) & ~1
    npairs = nfull // 2
    tails = _tail_sizes(epw - nfull * C)
    assert npairs >= 1
    rpt = NPAD // 16
    mesh = plsc.VectorSubcoreMesh(core_axis_name="c", subcore_axis_name="s")

    tail_scratch = [pltpu.VMEM((t,), jnp.int32) for t in tails]

    @functools.partial(
        pl.kernel,
        out_type=jax.ShapeDtypeStruct((2, NPAD, D), jnp.float32),
        mesh=mesh,
        scratch_types=[
            pltpu.VMEM((C,), jnp.int32),
            pltpu.VMEM((C,), jnp.int32),
            pltpu.VMEM((C, D), jnp.float32),
            pltpu.VMEM((C, D), jnp.float32),
            pltpu.VMEM_SHARED((NPAD, D), jnp.float32),
            pltpu.SemaphoreType.DMA,
            pltpu.SemaphoreType.DMA,
        ] + tail_scratch,
    )
    def scatter_k(em_hbm, dst_hbm, zeros_hbm, out_hbm,
                  idx0, idx1, rows0, rows1, agg, semA0, semA1, *tscratch):
        IDX = [idx0, idx1]
        ROWS = [rows0, rows1]
        SEMA = [semA0, semA1]
        c = lax.axis_index("c")
        s = lax.axis_index("s")
        wid = s * 2 + c
        lbase = wid * epw
        gbase = base + lbase
        pltpu.sync_copy(zeros_hbm.at[pl.ds(s * rpt, rpt)],
                        agg.at[pl.ds(s * rpt, rpt)])
        plsc.subcore_barrier()

        def a_descs(b, j):
            return (pltpu.make_async_copy(
                        dst_hbm.at[pl.ds(gbase + j * C, C)], IDX[b], SEMA[b]),
                    pltpu.make_async_copy(
                        em_hbm.at[pl.ds(lbase + j * C, C)], ROWS[b], SEMA[b]))

        for b in (0, 1):
            for d in a_descs(b, b):
                d.start()

        def body(j2, carry):
            j0 = 2 * j2
            for b in (0, 1):
                for d in a_descs(b, j0 + b):
                    d.wait()
                pltpu.sync_copy(ROWS[b], agg.at[IDX[b]], add=True)

                @pl.when(j2 < npairs - 1)
                def _(b=b, j0=j0):
                    for d in a_descs(b, j0 + 2 + b):
                        d.start()

            return carry

        lax.fori_loop(0, npairs, body, 0)

        toff = nfull * C
        for i, t in enumerate(tails):
            tidx = tscratch[i]
            pltpu.sync_copy(dst_hbm.at[pl.ds(gbase + toff, t)], tidx)
            pltpu.sync_copy(em_hbm.at[pl.ds(lbase + toff, t)],
                            rows0.at[pl.ds(0, t)])
            pltpu.sync_copy(rows0.at[pl.ds(0, t)], agg.at[tidx], add=True)
            toff += t

        plsc.subcore_barrier()
        pltpu.sync_copy(agg.at[pl.ds(s * rpt, rpt)],
                        out_hbm.at[c].at[pl.ds(s * rpt, rpt)])

    return scatter_k


def _pre_body(x_ref, wb_ref, wc_ref, b1_ref, p_ref, q_ref):
    xb = x_ref[...]
    p_ref[...] = jnp.dot(xb, wb_ref[...], preferred_element_type=jnp.float32)
    q_ref[...] = (jnp.dot(xb, wc_ref[...], preferred_element_type=jnp.float32)
                  + b1_ref[...])


def _edge_math(ea_ref, g_ref, wa_ref, w2_ref, b2_ref, eo_ref, em_ref):
    ea = ea_ref[...]
    h = jnp.maximum(
        jnp.dot(ea, wa_ref[...], preferred_element_type=jnp.float32)
        + g_ref[...], 0.0)
    em = jnp.dot(h, w2_ref[...], preferred_element_type=jnp.float32) + b2_ref[...]
    em_ref[...] = em
    eo_ref[...] = ea + em


def _edge_body(ea_ref, g_ref, wa_ref, w2_ref, b2_ref, eo_ref, em_ref):
    _edge_math(ea_ref, g_ref, wa_ref, w2_ref, b2_ref, eo_ref, em_ref)


def _edge_body_alias(ea_ref, g_ref, wa_ref, w2_ref, b2_ref,
                     eo_prev_ref, eo_ref, em_ref):
    del eo_prev_ref
    _edge_math(ea_ref, g_ref, wa_ref, w2_ref, b2_ref, eo_ref, em_ref)


def _make_node_body(n_parts):
    def body(*refs):
        x_ref = refs[0]
        parts = refs[1:1 + n_parts]
        w1x_ref, w1a_ref, b1_ref, w2_ref, b2_ref, xo_ref = refs[1 + n_parts:]
        xb = x_ref[...]
        agg = parts[0][0]
        for r in parts[1:]:
            agg = agg + r[0]
        h = jnp.maximum(
            jnp.dot(xb, w1x_ref[...], preferred_element_type=jnp.float32)
            + jnp.dot(agg, w1a_ref[...], preferred_element_type=jnp.float32)
            + b1_ref[...], 0.0)
        xo_ref[...] = (xb
                       + jnp.dot(h, w2_ref[...], preferred_element_type=jnp.float32)
                       + b2_ref[...])
    return body


def _fixed(shape):
    return pl.BlockSpec(shape, lambda i: (0,) * len(shape))


def kernel(x, edge_attr, edge_index, We1, be1, We2, be2, Wn1, bn1, Wn2, bn2):
    N, D = x.shape
    E = edge_attr.shape[0]
    src = edge_index[0]
    dst = edge_index[1]
    b1 = be1.reshape(1, D)
    b2 = be2.reshape(1, D)
    nb1 = bn1.reshape(1, D)
    nb2 = bn2.reshape(1, D)

    BN = 1000
    P, Q = pl.pallas_call(
        _pre_body,
        grid=(N // BN,),
        in_specs=[
            pl.BlockSpec((BN, D), lambda i: (i, 0)),
            _fixed((D, D)),
            _fixed((D, D)),
            _fixed((1, D)),
        ],
        out_specs=[
            pl.BlockSpec((BN, D), lambda i: (i, 0)),
            pl.BlockSpec((BN, D), lambda i: (i, 0)),
        ],
        out_shape=[
            jax.ShapeDtypeStruct((N, D), jnp.float32),
            jax.ShapeDtypeStruct((N, D), jnp.float32),
        ],
    )(x, We1[D:2 * D], We1[2 * D:3 * D], b1)

    NPAD = ((N + 127) // 128) * 128
    zeros = jnp.zeros((NPAD, D), jnp.float32)
    BE = 3200
    sizes = [E // K] * K
    bases = [sum(sizes[:k]) for k in range(K)]

    Gs = [_make_gather(N, E, D, bases[k], sizes[k])(P, Q, src, dst)
          for k in range(K)]

    eo = None
    EMs = []
    for k in range(K):
        boff = bases[k] // BE
        nblk = sizes[k] // BE
        ea_spec = pl.BlockSpec((BE, D), lambda i, boff=boff: (i + boff, 0))
        eo_spec = pl.BlockSpec((BE, D), lambda i, boff=boff: (i + boff, 0))
        common_in = [
            ea_spec,
            pl.BlockSpec((BE, D), lambda i: (i, 0)),
            _fixed((D, D)),
            _fixed((D, D)),
            _fixed((1, D)),
        ]
        out_shape = [
            jax.ShapeDtypeStruct((E, D), jnp.float32),
            jax.ShapeDtypeStruct((sizes[k], D), jnp.float32),
        ]
        out_specs = [eo_spec, pl.BlockSpec((BE, D), lambda i: (i, 0))]
        if k == 0:
            eo, em = pl.pallas_call(
                _edge_body,
                grid=(nblk,),
                in_specs=common_in,
                out_specs=out_specs,
                out_shape=out_shape,
            )(edge_attr, Gs[k], We1[:D], We2, b2)
        else:
            eo, em = pl.pallas_call(
                _edge_body_alias,
                grid=(nblk,),
                in_specs=common_in + [pl.BlockSpec(memory_space=pl.ANY)],
                out_specs=out_specs,
                out_shape=out_shape,
                input_output_aliases={5: 0},
            )(edge_attr, Gs[k], We1[:D], We2, b2, eo)
        EMs.append(em)

    partials = [_make_scatter(NPAD, E, D, bases[k], sizes[k])(EMs[k], dst, zeros)
                for k in range(K)]

    n_parts = 2 * K
    part_args = []
    part_specs = []
    for par in partials:
        for c in (0, 1):
            part_args.append(par)
            part_specs.append(pl.BlockSpec((1, BN, D), lambda i, c=c: (c, i, 0)))
    XO = pl.pallas_call(
        _make_node_body(n_parts),
        grid=(N // BN,),
        in_specs=[pl.BlockSpec((BN, D), lambda i: (i, 0))] + part_specs + [
            _fixed((D, D)),
            _fixed((D, D)),
            _fixed((1, D)),
            _fixed((D, D)),
            _fixed((1, D)),
        ],
        out_specs=pl.BlockSpec((BN, D), lambda i: (i, 0)),
        out_shape=jax.ShapeDtypeStruct((N, D), jnp.float32),
    )(x, *part_args, Wn1[:D], Wn1[D:], nb1, Wn2, nb2)

    return (XO, eo)


# speedup vs baseline: 1.0159x; 1.0017x over previous
import functools

import jax
import jax.numpy as jnp
from jax import lax
from jax.experimental import pallas as pl
from jax.experimental.pallas import tpu as pltpu
from jax.experimental.pallas import tpu_sc as plsc

NW = 32
C = 128
K = 2


def _tail_sizes(rem):
    out = []
    while rem:
        t = min(rem, C)
        out.append(t)
        rem -= t
    return out


def _make_gather(N, E, D, base, Ec):
    epw = Ec // NW
    nfull = (epw // C) & ~1
    npairs = nfull // 2
    tails = _tail_sizes(epw - nfull * C)
    assert npairs >= 1
    mesh = plsc.VectorSubcoreMesh(core_axis_name="c", subcore_axis_name="s")

    @functools.partial(
        pl.kernel,
        out_type=jax.ShapeDtypeStruct((Ec, D), jnp.float32),
        mesh=mesh,
        scratch_types=[
            pltpu.VMEM((C,), jnp.int32),
            pltpu.VMEM((C,), jnp.int32),
            pltpu.VMEM((C,), jnp.int32),
            pltpu.VMEM((C,), jnp.int32),
            pltpu.VMEM((C, D), jnp.float32),
            pltpu.VMEM((C, D), jnp.float32),
            pltpu.VMEM((C, D), jnp.float32),
            pltpu.VMEM((C, D), jnp.float32),
            pltpu.SemaphoreType.DMA,
            pltpu.SemaphoreType.DMA,
            pltpu.SemaphoreType.DMA,
            pltpu.SemaphoreType.DMA,
            pltpu.SemaphoreType.DMA,
            pltpu.SemaphoreType.DMA,
        ],
    )
    def gather_k(p_hbm, q_hbm, src_hbm, dst_hbm, g_hbm,
                 sidx0, sidx1, didx0, didx1, bs0, bs1, bd0, bd1,
                 semA0, semA1, semB0, semB1, semC0, semC1):
        SIDX = [sidx0, sidx1]
        DIDX = [didx0, didx1]
        BS = [bs0, bs1]
        BD = [bd0, bd1]
        SEMA = [semA0, semA1]
        SEMB = [semB0, semB1]
        SEMC = [semC0, semC1]
        wid = lax.axis_index("s") * 2 + lax.axis_index("c")
        lbase = wid * epw
        gbase = base + lbase

        def a_descs(b, j):
            off = gbase + j * C
            return (pltpu.make_async_copy(src_hbm.at[pl.ds(off, C)], SIDX[b], SEMA[b]),
                    pltpu.make_async_copy(dst_hbm.at[pl.ds(off, C)], DIDX[b], SEMA[b]))

        def b_descs(b):
            return (pltpu.make_async_copy(p_hbm.at[SIDX[b]], BS[b], SEMB[b]),
                    pltpu.make_async_copy(q_hbm.at[DIDX[b]], BD[b], SEMB[b]))

        def c_descs(b, j):
            off = lbase + j * C
            return (pltpu.make_async_copy(BS[b], g_hbm.at[pl.ds(off, C)], SEMC[b]),)

        def add_rows(dst_ref, src_ref, nrows):
            def row(i, carry):
                for k in range(D // 16):
                    plsc.addupdate(dst_ref.at[i, pl.ds(16 * k, 16)],
                                   src_ref[i, pl.ds(16 * k, 16)])
                return carry
            lax.fori_loop(0, nrows, row, 0)

        def start(descs):
            for d in descs:
                d.start()

        def wait(descs):
            for d in descs:
                d.wait()

        start(a_descs(0, 0))
        start(a_descs(1, 1))

        def body(j2, carry):
            j0 = 2 * j2
            for b in (0, 1):
                wait(a_descs(b, j0 + b))

                @pl.when(j2 > 0)
                def _(b=b, j0=j0):
                    wait(c_descs(b, j0 + b - 2))

                start(b_descs(b))
            for b in (0, 1):
                wait(b_descs(b))
                add_rows(BS[b], BD[b], C)
                start(c_descs(b, j0 + b))

                @pl.when(j2 < npairs - 1)
                def _(b=b, j0=j0):
                    start(a_descs(b, j0 + 2 + b))

            return carry

        lax.fori_loop(0, npairs, body, 0)
        wait(c_descs(0, nfull - 2))
        wait(c_descs(1, nfull - 1))

        toff = nfull * C
        for t in tails:
            pltpu.sync_copy(src_hbm.at[pl.ds(gbase + toff, t)],
                            sidx0.at[pl.ds(0, t)])
            pltpu.sync_copy(dst_hbm.at[pl.ds(gbase + toff, t)],
                            didx0.at[pl.ds(0, t)])
            tb = (pltpu.make_async_copy(
                      p_hbm.at[sidx0.at[pl.ds(0, t)]], bs0.at[pl.ds(0, t)], semB0),
                  pltpu.make_async_copy(
                      q_hbm.at[didx0.at[pl.ds(0, t)]], bd0.at[pl.ds(0, t)], semB1))
            start(tb)
            wait(tb)
            add_rows(bs0, bd0, t)
            pltpu.sync_copy(bs0.at[pl.ds(0, t)],
                            g_hbm.at[pl.ds(lbase + toff, t)])
            toff += t

    return gather_k


def _make_scatter(NPAD, E, D, base, Ec):
    epw = Ec // NW
    nfull = (epw // C) & ~1
    npairs = nfull // 2
    tails = _tail_sizes(epw - nfull * C)
    assert npairs >= 1
    rpt = NPAD // 16
    mesh = plsc.VectorSubcoreMesh(core_axis_name="c", subcore_axis_name="s")

    tail_scratch = [pltpu.VMEM((t,), jnp.int32) for t in tails]

    @functools.partial(
        pl.kernel,
        out_type=jax.ShapeDtypeStruct((2, NPAD, D), jnp.float32),
        mesh=mesh,
        scratch_types=[
            pltpu.VMEM((C,), jnp.int32),
            pltpu.VMEM((C,), jnp.int32),
            pltpu.VMEM((C, D), jnp.float32),
            pltpu.VMEM((C, D), jnp.float32),
            pltpu.VMEM_SHARED((NPAD, D), jnp.float32),
            pltpu.SemaphoreType.DMA,
            pltpu.SemaphoreType.DMA,
        ] + tail_scratch,
    )
    def scatter_k(em_hbm, dst_hbm, zeros_hbm, out_hbm,
                  idx0, idx1, rows0, rows1, agg, semA0, semA1, *tscratch):
        IDX = [idx0, idx1]
        ROWS = [rows0, rows1]
        SEMA = [semA0, semA1]
        c = lax.axis_index("c")
        s = lax.axis_index("s")
        wid = s * 2 + c
        lbase = wid * epw
        gbase = base + lbase
        pltpu.sync_copy(zeros_hbm.at[pl.ds(s * rpt, rpt)],
                        agg.at[pl.ds(s * rpt, rpt)])
        plsc.subcore_barrier()

        def a_descs(b, j):
            return (pltpu.make_async_copy(
                        dst_hbm.at[pl.ds(gbase + j * C, C)], IDX[b], SEMA[b]),
                    pltpu.make_async_copy(
                        em_hbm.at[pl.ds(lbase + j * C, C)], ROWS[b], SEMA[b]))

        for b in (0, 1):
            for d in a_descs(b, b):
                d.start()

        def body(j2, carry):
            j0 = 2 * j2
            for b in (0, 1):
                for d in a_descs(b, j0 + b):
                    d.wait()
                pltpu.sync_copy(ROWS[b], agg.at[IDX[b]], add=True)

                @pl.when(j2 < npairs - 1)
                def _(b=b, j0=j0):
                    for d in a_descs(b, j0 + 2 + b):
                        d.start()

            return carry

        lax.fori_loop(0, npairs, body, 0)

        toff = nfull * C
        for i, t in enumerate(tails):
            tidx = tscratch[i]
            pltpu.sync_copy(dst_hbm.at[pl.ds(gbase + toff, t)], tidx)
            pltpu.sync_copy(em_hbm.at[pl.ds(lbase + toff, t)],
                            rows0.at[pl.ds(0, t)])
            pltpu.sync_copy(rows0.at[pl.ds(0, t)], agg.at[tidx], add=True)
            toff += t

        plsc.subcore_barrier()
        pltpu.sync_copy(agg.at[pl.ds(s * rpt, rpt)],
                        out_hbm.at[c].at[pl.ds(s * rpt, rpt)])

    return scatter_k


def _pre_body(x_ref, wb_ref, wc_ref, b1_ref, p_ref, q_ref):
    xb = x_ref[...]
    p_ref[...] = jnp.dot(xb, wb_ref[...], preferred_element_type=jnp.float32)
    q_ref[...] = (jnp.dot(xb, wc_ref[...], preferred_element_type=jnp.float32)
                  + b1_ref[...])


def _edge_math(ea_ref, g_ref, wa_ref, w2_ref, b2_ref, eo_ref, em_ref):
    ea = ea_ref[...]
    h = jnp.maximum(
        jnp.dot(ea, wa_ref[...], preferred_element_type=jnp.float32)
        + g_ref[...], 0.0)
    em = jnp.dot(h, w2_ref[...], preferred_element_type=jnp.float32) + b2_ref[...]
    em_ref[...] = em
    eo_ref[...] = ea + em


def _edge_body(ea_ref, g_ref, wa_ref, w2_ref, b2_ref, eo_ref, em_ref):
    _edge_math(ea_ref, g_ref, wa_ref, w2_ref, b2_ref, eo_ref, em_ref)


def _edge_body_alias(ea_ref, g_ref, wa_ref, w2_ref, b2_ref,
                     eo_prev_ref, eo_ref, em_ref):
    del eo_prev_ref
    _edge_math(ea_ref, g_ref, wa_ref, w2_ref, b2_ref, eo_ref, em_ref)


def _make_node_body(n_parts):
    def body(*refs):
        x_ref = refs[0]
        parts = refs[1:1 + n_parts]
        w1x_ref, w1a_ref, b1_ref, w2_ref, b2_ref, xo_ref = refs[1 + n_parts:]
        xb = x_ref[...]
        agg = parts[0][0]
        for r in parts[1:]:
            agg = agg + r[0]
        h = jnp.maximum(
            jnp.dot(xb, w1x_ref[...], preferred_element_type=jnp.float32)
            + jnp.dot(agg, w1a_ref[...], preferred_element_type=jnp.float32)
            + b1_ref[...], 0.0)
        xo_ref[...] = (xb
                       + jnp.dot(h, w2_ref[...], preferred_element_type=jnp.float32)
                       + b2_ref[...])
    return body


def _fixed(shape):
    return pl.BlockSpec(shape, lambda i: (0,) * len(shape))


def kernel(x, edge_attr, edge_index, We1, be1, We2, be2, Wn1, bn1, Wn2, bn2):
    N, D = x.shape
    E = edge_attr.shape[0]
    src = edge_index[0]
    dst = edge_index[1]
    b1 = be1.reshape(1, D)
    b2 = be2.reshape(1, D)
    nb1 = bn1.reshape(1, D)
    nb2 = bn2.reshape(1, D)

    BN = 1000
    P, Q = pl.pallas_call(
        _pre_body,
        grid=(N // BN,),
        in_specs=[
            pl.BlockSpec((BN, D), lambda i: (i, 0)),
            pl.BlockSpec((D, D), lambda i: (1, 0)),
            pl.BlockSpec((D, D), lambda i: (2, 0)),
            _fixed((1, D)),
        ],
        out_specs=[
            pl.BlockSpec((BN, D), lambda i: (i, 0)),
            pl.BlockSpec((BN, D), lambda i: (i, 0)),
        ],
        out_shape=[
            jax.ShapeDtypeStruct((N, D), jnp.float32),
            jax.ShapeDtypeStruct((N, D), jnp.float32),
        ],
    )(x, We1, We1, b1)

    NPAD = ((N + 127) // 128) * 128
    zeros = jnp.zeros((NPAD, D), jnp.float32)
    BE = 3200
    sizes = [E // K] * K
    bases = [sum(sizes[:k]) for k in range(K)]

    Gs = [_make_gather(N, E, D, bases[k], sizes[k])(P, Q, src, dst)
          for k in range(K)]

    eo = None
    EMs = []
    for k in range(K):
        boff = bases[k] // BE
        nblk = sizes[k] // BE
        ea_spec = pl.BlockSpec((BE, D), lambda i, boff=boff: (i + boff, 0))
        eo_spec = pl.BlockSpec((BE, D), lambda i, boff=boff: (i + boff, 0))
        common_in = [
            ea_spec,
            pl.BlockSpec((BE, D), lambda i: (i, 0)),
            pl.BlockSpec((D, D), lambda i: (0, 0)),
            _fixed((D, D)),
            _fixed((1, D)),
        ]
        out_shape = [
            jax.ShapeDtypeStruct((E, D), jnp.float32),
            jax.ShapeDtypeStruct((sizes[k], D), jnp.float32),
        ]
        out_specs = [eo_spec, pl.BlockSpec((BE, D), lambda i: (i, 0))]
        if k == 0:
            eo, em = pl.pallas_call(
                _edge_body,
                grid=(nblk,),
                in_specs=common_in,
                out_specs=out_specs,
                out_shape=out_shape,
            )(edge_attr, Gs[k], We1, We2, b2)
        else:
            eo, em = pl.pallas_call(
                _edge_body_alias,
                grid=(nblk,),
                in_specs=common_in + [pl.BlockSpec(memory_space=pl.ANY)],
                out_specs=out_specs,
                out_shape=out_shape,
                input_output_aliases={5: 0},
            )(edge_attr, Gs[k], We1, We2, b2, eo)
        EMs.append(em)

    partials = [_make_scatter(NPAD, E, D, bases[k], sizes[k])(EMs[k], dst, zeros)
                for k in range(K)]

    n_parts = 2 * K
    part_args = []
    part_specs = []
    for par in partials:
        for c in (0, 1):
            part_args.append(par)
            part_specs.append(pl.BlockSpec((1, BN, D), lambda i, c=c: (c, i, 0)))
    XO = pl.pallas_call(
        _make_node_body(n_parts),
        grid=(N // BN,),
        in_specs=[pl.BlockSpec((BN, D), lambda i: (i, 0))] + part_specs + [
            pl.BlockSpec((D, D), lambda i: (0, 0)),
            pl.BlockSpec((D, D), lambda i: (1, 0)),
            _fixed((1, D)),
            _fixed((D, D)),
            _fixed((1, D)),
        ],
        out_specs=pl.BlockSpec((BN, D), lambda i: (i, 0)),
        out_shape=jax.ShapeDtypeStruct((N, D), jnp.float32),
    )(x, *part_args, Wn1, Wn1, nb1, Wn2, nb2)

    return (XO, eo)
